# Initial kernel scaffold; baseline (speedup 1.0000x reference)
import jax
import jax.numpy as jnp
from jax.experimental import pallas as pl


def kernel(x, edge_index, W1_0, b1_0, W2_0, b2_0, W1_1, b1_1, W2_1, b2_1, W1_2, b1_2, W2_2, b2_2, Wc, bc):
    raise NotImplementedError("write your pallas kernel here")


# speedup vs baseline: 3.1314x; 3.1314x over previous
import jax
import jax.numpy as jnp
from jax import lax
from jax.experimental import pallas as pl
from jax.experimental.pallas import tpu as pltpu
from jax.experimental.pallas import tpu_sc as plsc

N_NODES = 10000
N_EDGES = 320000
HID = 128
OUT_CH = 64

NC, NS = 2, 16
NW = NC * NS
CHUNK = 128
CHUNKS_PER_W = 80
E_PAD = NW * CHUNKS_PER_W * CHUNK
ACC_ROWS = 10112
DUMMY_ROW = N_NODES
PER_TILE = ACC_ROWS // NS

ROWB = 2000


def _sc_agg_body(h_hbm, isrc_hbm, idst_hbm, part_hbm,
                 isrc_v, idst_v, rows_v, acc_sh, sem):
    c = lax.axis_index("c")
    s = lax.axis_index("s")
    wid = c * NS + s

    pltpu.sync_copy(isrc_hbm.at[wid], isrc_v)
    pltpu.sync_copy(idst_hbm.at[wid], idst_v)

    def _zrow(i, carry):
        for k in range(HID // 16):
            rows_v[i, pl.ds(k * 16, 16)] = jnp.zeros((16,), jnp.float32)
        return carry
    lax.fori_loop(0, CHUNK, _zrow, 0)
    base = s * PER_TILE
    for r in range(PER_TILE // CHUNK):
        pltpu.sync_copy(rows_v, acc_sh.at[pl.ds(base + r * CHUNK, CHUNK)])
    rem = PER_TILE % CHUNK
    if rem:
        pltpu.sync_copy(
            rows_v.at[pl.ds(0, rem)],
            acc_sh.at[pl.ds(base + (PER_TILE // CHUNK) * CHUNK, rem)])
    plsc.subcore_barrier()

    def _edge_chunk(j, carry):
        pltpu.async_copy(h_hbm.at[isrc_v.at[j]], rows_v, sem).wait()
        pltpu.sync_copy(rows_v, acc_sh.at[idst_v.at[j]], add=True)
        return carry
    lax.fori_loop(0, CHUNKS_PER_W, _edge_chunk, 0)
    plsc.subcore_barrier()

    pltpu.sync_copy(acc_sh.at[pl.ds(s * PER_TILE, PER_TILE)],
                    part_hbm.at[c, pl.ds(s * PER_TILE, PER_TILE)])


_sc_agg = pl.kernel(
    _sc_agg_body,
    out_type=jax.ShapeDtypeStruct((NC, ACC_ROWS, HID), jnp.float32),
    mesh=plsc.VectorSubcoreMesh(core_axis_name="c", subcore_axis_name="s",
                                num_cores=NC, num_subcores=NS),
    scratch_types=[
        pltpu.VMEM((CHUNKS_PER_W, CHUNK), jnp.int32),
        pltpu.VMEM((CHUNKS_PER_W, CHUNK), jnp.int32),
        pltpu.VMEM((CHUNK, HID), jnp.float32),
        pltpu.VMEM_SHARED((ACC_ROWS, HID), jnp.float32),
        pltpu.SemaphoreType.DMA,
    ],
)


def _mlp_body(h_ref, p_ref, w1_ref, b1_ref, w2_ref, b2_ref, out_ref):
    a = h_ref[...] + p_ref[0] + p_ref[1]
    t = jnp.dot(a, w1_ref[...], preferred_element_type=jnp.float32) + b1_ref[...]
    t = jnp.maximum(t, 0.0)
    u = jnp.dot(t, w2_ref[...], preferred_element_type=jnp.float32) + b2_ref[...]
    out_ref[...] = jnp.maximum(u, 0.0)


def _mlp_final_body(h_ref, p_ref, w1_ref, b1_ref, w2_ref, b2_ref,
                    wc_ref, bc_ref, out_ref):
    a = h_ref[...] + p_ref[0] + p_ref[1]
    t = jnp.dot(a, w1_ref[...], preferred_element_type=jnp.float32) + b1_ref[...]
    t = jnp.maximum(t, 0.0)
    u = jnp.dot(t, w2_ref[...], preferred_element_type=jnp.float32) + b2_ref[...]
    u = jnp.maximum(u, 0.0)
    out_ref[...] = (jnp.dot(u, wc_ref[...], preferred_element_type=jnp.float32)
                    + bc_ref[...])


def _row_spec(d):
    return pl.BlockSpec((ROWB, d), lambda i: (i, 0))


def _full_spec(shape):
    nd = len(shape)
    return pl.BlockSpec(shape, lambda i: (0,) * nd)


_mlp = pl.pallas_call(
    _mlp_body,
    grid=(N_NODES // ROWB,),
    in_specs=[
        _row_spec(HID),
        pl.BlockSpec((NC, ROWB, HID), lambda i: (0, i, 0)),
        _full_spec((HID, HID)),
        _full_spec((1, HID)),
        _full_spec((HID, HID)),
        _full_spec((1, HID)),
    ],
    out_specs=_row_spec(HID),
    out_shape=jax.ShapeDtypeStruct((N_NODES, HID), jnp.float32),
)

_mlp_final = pl.pallas_call(
    _mlp_final_body,
    grid=(N_NODES // ROWB,),
    in_specs=[
        _row_spec(HID),
        pl.BlockSpec((NC, ROWB, HID), lambda i: (0, i, 0)),
        _full_spec((HID, HID)),
        _full_spec((1, HID)),
        _full_spec((HID, HID)),
        _full_spec((1, HID)),
        _full_spec((HID, OUT_CH)),
        _full_spec((1, OUT_CH)),
    ],
    out_specs=_row_spec(OUT_CH),
    out_shape=jax.ShapeDtypeStruct((N_NODES, OUT_CH), jnp.float32),
)


def kernel(x, edge_index, W1_0, b1_0, W2_0, b2_0, W1_1, b1_1, W2_1, b2_1,
           W1_2, b1_2, W2_2, b2_2, Wc, bc):
    ei = edge_index.astype(jnp.int32)
    pad = E_PAD - N_EDGES
    src_p = jnp.concatenate(
        [ei[0], jnp.zeros((pad,), jnp.int32)]).reshape(NW, CHUNKS_PER_W, CHUNK)
    dst_p = jnp.concatenate(
        [ei[1], jnp.full((pad,), DUMMY_ROW, jnp.int32)]
    ).reshape(NW, CHUNKS_PER_W, CHUNK)
    src_p, dst_p = lax.optimization_barrier((src_p, dst_p))

    params = [(W1_0, b1_0, W2_0, b2_0), (W1_1, b1_1, W2_1, b2_1),
              (W1_2, b1_2, W2_2, b2_2)]
    h = x
    for i, (W1, b1, W2, b2) in enumerate(params):
        part = _sc_agg(h, src_p, dst_p)
        if i < 2:
            h = _mlp(h, part, W1, b1.reshape(1, HID), W2, b2.reshape(1, HID))
        else:
            out = _mlp_final(h, part, W1, b1.reshape(1, HID), W2,
                             b2.reshape(1, HID), Wc, bc.reshape(1, OUT_CH))
    return out


# speedup vs baseline: 3.3859x; 1.0813x over previous
import jax
import jax.numpy as jnp
from jax import lax
from jax.experimental import pallas as pl
from jax.experimental.pallas import tpu as pltpu
from jax.experimental.pallas import tpu_sc as plsc

N_NODES = 10000
N_EDGES = 320000
HID = 128
OUT_CH = 64

NC, NS = 2, 16
NW = NC * NS
CHUNK = 128
CHUNKS_PER_W = 80
HALF_CHUNKS = CHUNKS_PER_W // 2
E_PAD = NW * CHUNKS_PER_W * CHUNK
ACC_ROWS = 10112
DUMMY_ROW = N_NODES
PER_TILE = ACC_ROWS // NS

ROWB = 2000


def _sc_agg_body(h_hbm, isrc_hbm, idst_hbm, part_hbm,
                 isrc_v, idst_v, rows_a, rows_b, acc_sh,
                 sem_ga, sem_gb, sem_sa, sem_sb):
    c = lax.axis_index("c")
    s = lax.axis_index("s")
    wid = c * NS + s

    def _zrow(i, carry):
        for k in range(HID // 16):
            rows_a[i, pl.ds(k * 16, 16)] = jnp.zeros((16,), jnp.float32)
        return carry
    lax.fori_loop(0, CHUNK, _zrow, 0)
    base = s * PER_TILE
    for r in range(PER_TILE // CHUNK):
        pltpu.sync_copy(rows_a, acc_sh.at[pl.ds(base + r * CHUNK, CHUNK)])
    rem = PER_TILE % CHUNK
    if rem:
        pltpu.sync_copy(
            rows_a.at[pl.ds(0, rem)],
            acc_sh.at[pl.ds(base + (PER_TILE // CHUNK) * CHUNK, rem)])
    plsc.subcore_barrier()

    for half in range(2):
        off = half * HALF_CHUNKS
        pltpu.sync_copy(isrc_hbm.at[wid, pl.ds(off, HALF_CHUNKS)], isrc_v)
        pltpu.sync_copy(idst_hbm.at[wid, pl.ds(off, HALF_CHUNKS)], idst_v)

        pltpu.async_copy(h_hbm.at[isrc_v.at[0]], rows_a, sem_ga)
        pltpu.async_copy(h_hbm.at[isrc_v.at[1]], rows_b, sem_gb)

        def _pair(k, carry):
            a = 2 * k
            b = a + 1
            pltpu.make_async_copy(h_hbm.at[isrc_v.at[a]], rows_a, sem_ga).wait()
            pltpu.async_copy(rows_a, acc_sh.at[idst_v.at[a]], sem_sa, add=True)
            pltpu.make_async_copy(h_hbm.at[isrc_v.at[b]], rows_b, sem_gb).wait()
            pltpu.async_copy(rows_b, acc_sh.at[idst_v.at[b]], sem_sb, add=True)

            @pl.when(a + 2 < HALF_CHUNKS)
            def _():
                pltpu.make_async_copy(
                    rows_a, acc_sh.at[idst_v.at[a]], sem_sa).wait()
                pltpu.async_copy(h_hbm.at[isrc_v.at[a + 2]], rows_a, sem_ga)

            @pl.when(b + 2 < HALF_CHUNKS)
            def _():
                pltpu.make_async_copy(
                    rows_b, acc_sh.at[idst_v.at[b]], sem_sb).wait()
                pltpu.async_copy(h_hbm.at[isrc_v.at[b + 2]], rows_b, sem_gb)
            return carry

        lax.fori_loop(0, HALF_CHUNKS // 2, _pair, 0)
        pltpu.make_async_copy(
            rows_a, acc_sh.at[idst_v.at[HALF_CHUNKS - 2]], sem_sa).wait()
        pltpu.make_async_copy(
            rows_b, acc_sh.at[idst_v.at[HALF_CHUNKS - 1]], sem_sb).wait()
    plsc.subcore_barrier()

    pltpu.sync_copy(acc_sh.at[pl.ds(s * PER_TILE, PER_TILE)],
                    part_hbm.at[c, pl.ds(s * PER_TILE, PER_TILE)])


_sc_agg = pl.kernel(
    _sc_agg_body,
    out_type=jax.ShapeDtypeStruct((NC, ACC_ROWS, HID), jnp.float32),
    mesh=plsc.VectorSubcoreMesh(core_axis_name="c", subcore_axis_name="s",
                                num_cores=NC, num_subcores=NS),
    scratch_types=[
        pltpu.VMEM((HALF_CHUNKS, CHUNK), jnp.int32),
        pltpu.VMEM((HALF_CHUNKS, CHUNK), jnp.int32),
        pltpu.VMEM((CHUNK, HID), jnp.float32),
        pltpu.VMEM((CHUNK, HID), jnp.float32),
        pltpu.VMEM_SHARED((ACC_ROWS, HID), jnp.float32),
        pltpu.SemaphoreType.DMA,
        pltpu.SemaphoreType.DMA,
        pltpu.SemaphoreType.DMA,
        pltpu.SemaphoreType.DMA,
    ],
)


def _mlp_body(h_ref, p_ref, w1_ref, b1_ref, w2_ref, b2_ref, out_ref):
    a = h_ref[...] + p_ref[0] + p_ref[1]
    t = jnp.dot(a, w1_ref[...], preferred_element_type=jnp.float32) + b1_ref[...]
    t = jnp.maximum(t, 0.0)
    u = jnp.dot(t, w2_ref[...], preferred_element_type=jnp.float32) + b2_ref[...]
    out_ref[...] = jnp.maximum(u, 0.0)


def _mlp_final_body(h_ref, p_ref, w1_ref, b1_ref, w2_ref, b2_ref,
                    wc_ref, bc_ref, out_ref):
    a = h_ref[...] + p_ref[0] + p_ref[1]
    t = jnp.dot(a, w1_ref[...], preferred_element_type=jnp.float32) + b1_ref[...]
    t = jnp.maximum(t, 0.0)
    u = jnp.dot(t, w2_ref[...], preferred_element_type=jnp.float32) + b2_ref[...]
    u = jnp.maximum(u, 0.0)
    out_ref[...] = (jnp.dot(u, wc_ref[...], preferred_element_type=jnp.float32)
                    + bc_ref[...])


def _row_spec(d):
    return pl.BlockSpec((ROWB, d), lambda i: (i, 0))


def _full_spec(shape):
    nd = len(shape)
    return pl.BlockSpec(shape, lambda i: (0,) * nd)


_mlp = pl.pallas_call(
    _mlp_body,
    grid=(N_NODES // ROWB,),
    in_specs=[
        _row_spec(HID),
        pl.BlockSpec((NC, ROWB, HID), lambda i: (0, i, 0)),
        _full_spec((HID, HID)),
        _full_spec((1, HID)),
        _full_spec((HID, HID)),
        _full_spec((1, HID)),
    ],
    out_specs=_row_spec(HID),
    out_shape=jax.ShapeDtypeStruct((N_NODES, HID), jnp.float32),
)

_mlp_final = pl.pallas_call(
    _mlp_final_body,
    grid=(N_NODES // ROWB,),
    in_specs=[
        _row_spec(HID),
        pl.BlockSpec((NC, ROWB, HID), lambda i: (0, i, 0)),
        _full_spec((HID, HID)),
        _full_spec((1, HID)),
        _full_spec((HID, HID)),
        _full_spec((1, HID)),
        _full_spec((HID, OUT_CH)),
        _full_spec((1, OUT_CH)),
    ],
    out_specs=_row_spec(OUT_CH),
    out_shape=jax.ShapeDtypeStruct((N_NODES, OUT_CH), jnp.float32),
)


def kernel(x, edge_index, W1_0, b1_0, W2_0, b2_0, W1_1, b1_1, W2_1, b2_1,
           W1_2, b1_2, W2_2, b2_2, Wc, bc):
    ei = edge_index.astype(jnp.int32)
    pad = E_PAD - N_EDGES
    src_p = jnp.concatenate(
        [ei[0], jnp.zeros((pad,), jnp.int32)]).reshape(NW, CHUNKS_PER_W, CHUNK)
    dst_p = jnp.concatenate(
        [ei[1], jnp.full((pad,), DUMMY_ROW, jnp.int32)]
    ).reshape(NW, CHUNKS_PER_W, CHUNK)
    src_p, dst_p = lax.optimization_barrier((src_p, dst_p))

    params = [(W1_0, b1_0, W2_0, b2_0), (W1_1, b1_1, W2_1, b2_1),
              (W1_2, b1_2, W2_2, b2_2)]
    h = x
    for i, (W1, b1, W2, b2) in enumerate(params):
        part = _sc_agg(h, src_p, dst_p)
        if i < 2:
            h = _mlp(h, part, W1, b1.reshape(1, HID), W2, b2.reshape(1, HID))
        else:
            out = _mlp_final(h, part, W1, b1.reshape(1, HID), W2,
                             b2.reshape(1, HID), Wc, bc.reshape(1, OUT_CH))
    return out


# speedup vs baseline: 9.9713x; 2.9449x over previous
import jax
import jax.numpy as jnp
from jax import lax
from jax.experimental import pallas as pl
from jax.experimental.pallas import tpu as pltpu
from jax.experimental.pallas import tpu_sc as plsc

N_NODES = 10000
N_EDGES = 320000
HID = 128
OUT_CH = 64

NC, NS = 2, 16
NW = NC * NS
CHUNK = 128
CHUNKS_PER_W = 80
HALF_CHUNKS = CHUNKS_PER_W // 2
E_PAD = NW * CHUNKS_PER_W * CHUNK
ACC_ROWS = 10112
DUMMY_ROW = N_NODES
PER_TILE = ACC_ROWS // NS

ROWB = 2000


def _sc_agg_body(h_hbm, isrc_hbm, idst_hbm, part_hbm,
                 isrc_v, idst_v, rows_a, rows_b, acc_sh,
                 sem_ga, sem_gb, sem_sa, sem_sb):
    c = lax.axis_index("c")
    s = lax.axis_index("s")
    wid = c * NS + s

    def _zrow(i, carry):
        for k in range(HID // 16):
            rows_a[i, pl.ds(k * 16, 16)] = jnp.zeros((16,), jnp.float32)
        return carry
    lax.fori_loop(0, CHUNK, _zrow, 0)
    base = s * PER_TILE
    for r in range(PER_TILE // CHUNK):
        pltpu.sync_copy(rows_a, acc_sh.at[pl.ds(base + r * CHUNK, CHUNK)])
    rem = PER_TILE % CHUNK
    if rem:
        pltpu.sync_copy(
            rows_a.at[pl.ds(0, rem)],
            acc_sh.at[pl.ds(base + (PER_TILE // CHUNK) * CHUNK, rem)])
    plsc.subcore_barrier()

    for half in range(2):
        off = half * HALF_CHUNKS
        pltpu.sync_copy(isrc_hbm.at[wid, pl.ds(off, HALF_CHUNKS)], isrc_v)
        pltpu.sync_copy(idst_hbm.at[wid, pl.ds(off, HALF_CHUNKS)], idst_v)

        pltpu.async_copy(h_hbm.at[isrc_v.at[0]], rows_a, sem_ga)
        pltpu.async_copy(h_hbm.at[isrc_v.at[1]], rows_b, sem_gb)

        def _pair(k, carry):
            a = 2 * k
            b = a + 1
            pltpu.make_async_copy(h_hbm.at[isrc_v.at[a]], rows_a, sem_ga).wait()
            pltpu.async_copy(rows_a, acc_sh.at[idst_v.at[a]], sem_sa, add=True)
            pltpu.make_async_copy(h_hbm.at[isrc_v.at[b]], rows_b, sem_gb).wait()
            pltpu.async_copy(rows_b, acc_sh.at[idst_v.at[b]], sem_sb, add=True)

            @pl.when(a + 2 < HALF_CHUNKS)
            def _():
                pltpu.make_async_copy(
                    rows_a, acc_sh.at[idst_v.at[a]], sem_sa).wait()
                pltpu.async_copy(h_hbm.at[isrc_v.at[a + 2]], rows_a, sem_ga)

            @pl.when(b + 2 < HALF_CHUNKS)
            def _():
                pltpu.make_async_copy(
                    rows_b, acc_sh.at[idst_v.at[b]], sem_sb).wait()
                pltpu.async_copy(h_hbm.at[isrc_v.at[b + 2]], rows_b, sem_gb)
            return carry

        lax.fori_loop(0, HALF_CHUNKS // 2, _pair, 0)
        pltpu.make_async_copy(
            rows_a, acc_sh.at[idst_v.at[HALF_CHUNKS - 2]], sem_sa).wait()
        pltpu.make_async_copy(
            rows_b, acc_sh.at[idst_v.at[HALF_CHUNKS - 1]], sem_sb).wait()
    plsc.subcore_barrier()

    pltpu.sync_copy(acc_sh.at[pl.ds(s * PER_TILE, PER_TILE)],
                    part_hbm.at[c, pl.ds(s * PER_TILE, PER_TILE)])


_sc_agg = pl.kernel(
    _sc_agg_body,
    out_type=jax.ShapeDtypeStruct((NC, ACC_ROWS, HID), jnp.float32),
    mesh=plsc.VectorSubcoreMesh(core_axis_name="c", subcore_axis_name="s",
                                num_cores=NC, num_subcores=NS),
    scratch_types=[
        pltpu.VMEM((HALF_CHUNKS, CHUNK), jnp.int32),
        pltpu.VMEM((HALF_CHUNKS, CHUNK), jnp.int32),
        pltpu.VMEM((CHUNK, HID), jnp.float32),
        pltpu.VMEM((CHUNK, HID), jnp.float32),
        pltpu.VMEM_SHARED((ACC_ROWS, HID), jnp.float32),
        pltpu.SemaphoreType.DMA,
        pltpu.SemaphoreType.DMA,
        pltpu.SemaphoreType.DMA,
        pltpu.SemaphoreType.DMA,
    ],
)


def _mlp_body(h_ref, p_ref, w1_ref, b1_ref, w2_ref, b2_ref, out_ref):
    a = h_ref[...] + p_ref[0] + p_ref[1]
    t = jnp.dot(a, w1_ref[...], preferred_element_type=jnp.float32) + b1_ref[...]
    t = jnp.maximum(t, 0.0)
    u = jnp.dot(t, w2_ref[...], preferred_element_type=jnp.float32) + b2_ref[...]
    out_ref[...] = jnp.maximum(u, 0.0)


def _mlp_final_body(h_ref, p_ref, w1_ref, b1_ref, w2_ref, b2_ref,
                    wc_ref, bc_ref, out_ref):
    a = h_ref[...] + p_ref[0] + p_ref[1]
    t = jnp.dot(a, w1_ref[...], preferred_element_type=jnp.float32) + b1_ref[...]
    t = jnp.maximum(t, 0.0)
    u = jnp.dot(t, w2_ref[...], preferred_element_type=jnp.float32) + b2_ref[...]
    u = jnp.maximum(u, 0.0)
    out_ref[...] = (jnp.dot(u, wc_ref[...], preferred_element_type=jnp.float32)
                    + bc_ref[...])


def _row_spec(d):
    return pl.BlockSpec((ROWB, d), lambda i: (i, 0))


def _full_spec(shape):
    nd = len(shape)
    return pl.BlockSpec(shape, lambda i: (0,) * nd)


_mlp = pl.pallas_call(
    _mlp_body,
    grid=(N_NODES // ROWB,),
    in_specs=[
        _row_spec(HID),
        pl.BlockSpec((NC, ROWB, HID), lambda i: (0, i, 0)),
        _full_spec((HID, HID)),
        _full_spec((1, HID)),
        _full_spec((HID, HID)),
        _full_spec((1, HID)),
    ],
    out_specs=_row_spec(HID),
    out_shape=jax.ShapeDtypeStruct((N_NODES, HID), jnp.float32),
)

_mlp_final = pl.pallas_call(
    _mlp_final_body,
    grid=(N_NODES // ROWB,),
    in_specs=[
        _row_spec(HID),
        pl.BlockSpec((NC, ROWB, HID), lambda i: (0, i, 0)),
        _full_spec((HID, HID)),
        _full_spec((1, HID)),
        _full_spec((HID, HID)),
        _full_spec((1, HID)),
        _full_spec((HID, OUT_CH)),
        _full_spec((1, OUT_CH)),
    ],
    out_specs=_row_spec(OUT_CH),
    out_shape=jax.ShapeDtypeStruct((N_NODES, OUT_CH), jnp.float32),
)


def kernel(x, edge_index, W1_0, b1_0, W2_0, b2_0, W1_1, b1_1, W2_1, b2_1,
           W1_2, b1_2, W2_2, b2_2, Wc, bc):
    ei = edge_index.astype(jnp.int32)
    pad = E_PAD - N_EDGES
    pad_src = jnp.arange(pad, dtype=jnp.int32) % N_NODES
    pad_dst = DUMMY_ROW + jnp.arange(pad, dtype=jnp.int32) % (ACC_ROWS - N_NODES)
    src_p = jnp.concatenate([ei[0], pad_src]).reshape(NW, CHUNKS_PER_W, CHUNK)
    dst_p = jnp.concatenate([ei[1], pad_dst]).reshape(NW, CHUNKS_PER_W, CHUNK)
    src_p, dst_p = lax.optimization_barrier((src_p, dst_p))

    params = [(W1_0, b1_0, W2_0, b2_0), (W1_1, b1_1, W2_1, b2_1),
              (W1_2, b1_2, W2_2, b2_2)]
    h = x
    for i, (W1, b1, W2, b2) in enumerate(params):
        part = _sc_agg(h, src_p, dst_p)
        if i < 2:
            h = _mlp(h, part, W1, b1.reshape(1, HID), W2, b2.reshape(1, HID))
        else:
            out = _mlp_final(h, part, W1, b1.reshape(1, HID), W2,
                             b2.reshape(1, HID), Wc, bc.reshape(1, OUT_CH))
    return out


# speedup vs baseline: 10.2520x; 1.0281x over previous
import jax
import jax.numpy as jnp
from jax import lax
from jax.experimental import pallas as pl
from jax.experimental.pallas import tpu as pltpu
from jax.experimental.pallas import tpu_sc as plsc

N_NODES = 10000
N_EDGES = 320000
HID = 128
OUT_CH = 64

NC, NS = 2, 16
NW = NC * NS
CHUNK = 128
CHUNKS_PER_W = 80
HALF_CHUNKS = CHUNKS_PER_W // 2
E_PAD = NW * CHUNKS_PER_W * CHUNK
ACC_ROWS = 10112
DUMMY_ROW = N_NODES
PER_TILE = ACC_ROWS // NS

ROWB = 2000


def _sc_agg_body(h_hbm, isrc_hbm, idst_hbm, part_hbm,
                 isrc_v, idst_v, rows_a, rows_b, acc_sh,
                 sem_ga, sem_gb, sem_sa, sem_sb):
    c = lax.axis_index("c")
    s = lax.axis_index("s")
    wid = c * NS + s

    pltpu.sync_copy(isrc_hbm.at[wid, pl.ds(0, HALF_CHUNKS)], isrc_v)
    pltpu.sync_copy(idst_hbm.at[wid, pl.ds(0, HALF_CHUNKS)], idst_v)
    pltpu.async_copy(h_hbm.at[isrc_v.at[0]], rows_a, sem_ga)

    def _zrow(i, carry):
        for k in range(HID // 16):
            rows_b[i, pl.ds(k * 16, 16)] = jnp.zeros((16,), jnp.float32)
        return carry
    lax.fori_loop(0, CHUNK, _zrow, 0)
    base = s * PER_TILE
    for r in range(PER_TILE // CHUNK):
        pltpu.sync_copy(rows_b, acc_sh.at[pl.ds(base + r * CHUNK, CHUNK)])
    rem = PER_TILE % CHUNK
    if rem:
        pltpu.sync_copy(
            rows_b.at[pl.ds(0, rem)],
            acc_sh.at[pl.ds(base + (PER_TILE // CHUNK) * CHUNK, rem)])
    pltpu.async_copy(h_hbm.at[isrc_v.at[1]], rows_b, sem_gb)
    plsc.subcore_barrier()

    for half in range(2):
        if half:
            pltpu.sync_copy(
                isrc_hbm.at[wid, pl.ds(HALF_CHUNKS, HALF_CHUNKS)], isrc_v)
            pltpu.sync_copy(
                idst_hbm.at[wid, pl.ds(HALF_CHUNKS, HALF_CHUNKS)], idst_v)
            pltpu.async_copy(h_hbm.at[isrc_v.at[0]], rows_a, sem_ga)
            pltpu.async_copy(h_hbm.at[isrc_v.at[1]], rows_b, sem_gb)

        def _pair(k, carry):
            a = 2 * k
            b = a + 1
            pltpu.make_async_copy(h_hbm.at[isrc_v.at[a]], rows_a, sem_ga).wait()
            pltpu.async_copy(rows_a, acc_sh.at[idst_v.at[a]], sem_sa, add=True)
            pltpu.make_async_copy(h_hbm.at[isrc_v.at[b]], rows_b, sem_gb).wait()
            pltpu.async_copy(rows_b, acc_sh.at[idst_v.at[b]], sem_sb, add=True)

            @pl.when(a + 2 < HALF_CHUNKS)
            def _():
                pltpu.make_async_copy(
                    rows_a, acc_sh.at[idst_v.at[a]], sem_sa).wait()
                pltpu.async_copy(h_hbm.at[isrc_v.at[a + 2]], rows_a, sem_ga)

            @pl.when(b + 2 < HALF_CHUNKS)
            def _():
                pltpu.make_async_copy(
                    rows_b, acc_sh.at[idst_v.at[b]], sem_sb).wait()
                pltpu.async_copy(h_hbm.at[isrc_v.at[b + 2]], rows_b, sem_gb)
            return carry

        lax.fori_loop(0, HALF_CHUNKS // 2, _pair, 0)
        pltpu.make_async_copy(
            rows_a, acc_sh.at[idst_v.at[HALF_CHUNKS - 2]], sem_sa).wait()
        pltpu.make_async_copy(
            rows_b, acc_sh.at[idst_v.at[HALF_CHUNKS - 1]], sem_sb).wait()
    plsc.subcore_barrier()

    pltpu.sync_copy(acc_sh.at[pl.ds(s * PER_TILE, PER_TILE)],
                    part_hbm.at[c, pl.ds(s * PER_TILE, PER_TILE)])


_sc_agg = pl.kernel(
    _sc_agg_body,
    out_type=jax.ShapeDtypeStruct((NC, ACC_ROWS, HID), jnp.float32),
    mesh=plsc.VectorSubcoreMesh(core_axis_name="c", subcore_axis_name="s",
                                num_cores=NC, num_subcores=NS),
    scratch_types=[
        pltpu.VMEM((HALF_CHUNKS, CHUNK), jnp.int32),
        pltpu.VMEM((HALF_CHUNKS, CHUNK), jnp.int32),
        pltpu.VMEM((CHUNK, HID), jnp.float32),
        pltpu.VMEM((CHUNK, HID), jnp.float32),
        pltpu.VMEM_SHARED((ACC_ROWS, HID), jnp.float32),
        pltpu.SemaphoreType.DMA,
        pltpu.SemaphoreType.DMA,
        pltpu.SemaphoreType.DMA,
        pltpu.SemaphoreType.DMA,
    ],
)


def _mlp_body(h_ref, p_ref, w1_ref, b1_ref, w2_ref, b2_ref, out_ref):
    a = h_ref[...] + p_ref[0] + p_ref[1]
    t = jnp.dot(a, w1_ref[...], preferred_element_type=jnp.float32) + b1_ref[...]
    t = jnp.maximum(t, 0.0)
    u = jnp.dot(t, w2_ref[...], preferred_element_type=jnp.float32) + b2_ref[...]
    out_ref[...] = jnp.maximum(u, 0.0)


def _mlp_final_body(h_ref, p_ref, w1_ref, b1_ref, w2_ref, b2_ref,
                    wc_ref, bc_ref, out_ref):
    a = h_ref[...] + p_ref[0] + p_ref[1]
    t = jnp.dot(a, w1_ref[...], preferred_element_type=jnp.float32) + b1_ref[...]
    t = jnp.maximum(t, 0.0)
    u = jnp.dot(t, w2_ref[...], preferred_element_type=jnp.float32) + b2_ref[...]
    u = jnp.maximum(u, 0.0)
    out_ref[...] = (jnp.dot(u, wc_ref[...], preferred_element_type=jnp.float32)
                    + bc_ref[...])


def _row_spec(d):
    return pl.BlockSpec((ROWB, d), lambda i: (i, 0))


def _full_spec(shape):
    nd = len(shape)
    return pl.BlockSpec(shape, lambda i: (0,) * nd)


_mlp = pl.pallas_call(
    _mlp_body,
    grid=(N_NODES // ROWB,),
    in_specs=[
        _row_spec(HID),
        pl.BlockSpec((NC, ROWB, HID), lambda i: (0, i, 0)),
        _full_spec((HID, HID)),
        _full_spec((1, HID)),
        _full_spec((HID, HID)),
        _full_spec((1, HID)),
    ],
    out_specs=_row_spec(HID),
    out_shape=jax.ShapeDtypeStruct((N_NODES, HID), jnp.float32),
)

_mlp_final = pl.pallas_call(
    _mlp_final_body,
    grid=(N_NODES // ROWB,),
    in_specs=[
        _row_spec(HID),
        pl.BlockSpec((NC, ROWB, HID), lambda i: (0, i, 0)),
        _full_spec((HID, HID)),
        _full_spec((1, HID)),
        _full_spec((HID, HID)),
        _full_spec((1, HID)),
        _full_spec((HID, OUT_CH)),
        _full_spec((1, OUT_CH)),
    ],
    out_specs=_row_spec(OUT_CH),
    out_shape=jax.ShapeDtypeStruct((N_NODES, OUT_CH), jnp.float32),
)


def kernel(x, edge_index, W1_0, b1_0, W2_0, b2_0, W1_1, b1_1, W2_1, b2_1,
           W1_2, b1_2, W2_2, b2_2, Wc, bc):
    ei = edge_index.astype(jnp.int32)
    pad = E_PAD - N_EDGES
    pad_src = jnp.arange(pad, dtype=jnp.int32) % N_NODES
    pad_dst = DUMMY_ROW + jnp.arange(pad, dtype=jnp.int32) % (ACC_ROWS - N_NODES)
    src_p = jnp.concatenate([ei[0], pad_src]).reshape(NW, CHUNKS_PER_W, CHUNK)
    dst_p = jnp.concatenate([ei[1], pad_dst]).reshape(NW, CHUNKS_PER_W, CHUNK)
    src_p, dst_p = lax.optimization_barrier((src_p, dst_p))

    params = [(W1_0, b1_0, W2_0, b2_0), (W1_1, b1_1, W2_1, b2_1),
              (W1_2, b1_2, W2_2, b2_2)]
    h = x
    for i, (W1, b1, W2, b2) in enumerate(params):
        part = _sc_agg(h, src_p, dst_p)
        if i < 2:
            h = _mlp(h, part, W1, b1.reshape(1, HID), W2, b2.reshape(1, HID))
        else:
            out = _mlp_final(h, part, W1, b1.reshape(1, HID), W2,
                             b2.reshape(1, HID), Wc, bc.reshape(1, OUT_CH))
    return out


# speedup vs baseline: 12.3426x; 1.2039x over previous
import jax
import jax.numpy as jnp
from jax import lax
from jax.experimental import pallas as pl
from jax.experimental.pallas import tpu as pltpu
from jax.experimental.pallas import tpu_sc as plsc

N_NODES = 10000
N_EDGES = 320000
HID = 128
OUT_CH = 64

NC, NS = 2, 16
NW = NC * NS
CHUNK = 64
IDXW = 128
IDX_ROWS = 80
HALF_ROWS = IDX_ROWS // 2
HALF_CHUNKS = HALF_ROWS * 2
E_PAD = NW * IDX_ROWS * IDXW
NBUF = 4
ACC_ROWS = 10112
DUMMY_ROW = N_NODES
PER_TILE = ACC_ROWS // NS

ROWB = 2000


def _sc_agg_body(h_hbm, isrc_hbm, idst_hbm, part_hbm,
                 isrc_v, idst_v, rows, sem_g, sem_s, acc_sh):
    c = lax.axis_index("c")
    s = lax.axis_index("s")
    wid = c * NS + s

    def _sidx(ref, k, b):
        return ref.at[2 * k + (b // 2), pl.ds(64 * (b % 2), 64)]

    pltpu.sync_copy(isrc_hbm.at[wid, pl.ds(0, HALF_ROWS)], isrc_v)
    pltpu.sync_copy(idst_hbm.at[wid, pl.ds(0, HALF_ROWS)], idst_v)
    for b in range(NBUF - 1):
        pltpu.async_copy(h_hbm.at[_sidx(isrc_v, 0, b)], rows[b], sem_g[b])

    zbuf = rows[NBUF - 1]

    def _zrow(i, carry):
        for k in range(HID // 16):
            zbuf[i, pl.ds(k * 16, 16)] = jnp.zeros((16,), jnp.float32)
        return carry
    lax.fori_loop(0, CHUNK, _zrow, 0)
    base = s * PER_TILE
    for r in range(PER_TILE // CHUNK):
        pltpu.sync_copy(zbuf, acc_sh.at[pl.ds(base + r * CHUNK, CHUNK)])
    rem = PER_TILE % CHUNK
    if rem:
        pltpu.sync_copy(
            zbuf.at[pl.ds(0, rem)],
            acc_sh.at[pl.ds(base + (PER_TILE // CHUNK) * CHUNK, rem)])
    pltpu.async_copy(
        h_hbm.at[_sidx(isrc_v, 0, NBUF - 1)], rows[NBUF - 1], sem_g[NBUF - 1])
    plsc.subcore_barrier()

    for half in range(2):
        if half:
            pltpu.sync_copy(
                isrc_hbm.at[wid, pl.ds(HALF_ROWS, HALF_ROWS)], isrc_v)
            pltpu.sync_copy(
                idst_hbm.at[wid, pl.ds(HALF_ROWS, HALF_ROWS)], idst_v)
            for b in range(NBUF):
                pltpu.async_copy(
                    h_hbm.at[_sidx(isrc_v, 0, b)], rows[b], sem_g[b])

        def _round(k, carry):
            for b in range(NBUF):
                pltpu.make_async_copy(
                    h_hbm.at[_sidx(isrc_v, k, b)], rows[b], sem_g[b]).wait()
                pltpu.async_copy(
                    rows[b], acc_sh.at[_sidx(idst_v, k, b)], sem_s[b],
                    add=True)
            for b in range(NBUF):
                @pl.when(NBUF * k + b + NBUF < HALF_CHUNKS)
                def _():
                    pltpu.make_async_copy(
                        rows[b], acc_sh.at[_sidx(idst_v, k, b)],
                        sem_s[b]).wait()
                    pltpu.async_copy(
                        h_hbm.at[_sidx(isrc_v, k + 1, b)], rows[b], sem_g[b])
            return carry

        lax.fori_loop(0, HALF_CHUNKS // NBUF, _round, 0)
        last = HALF_CHUNKS // NBUF - 1
        for b in range(NBUF):
            pltpu.make_async_copy(
                rows[b], acc_sh.at[_sidx(idst_v, last, b)], sem_s[b]).wait()
    plsc.subcore_barrier()

    pltpu.sync_copy(acc_sh.at[pl.ds(s * PER_TILE, PER_TILE)],
                    part_hbm.at[c, pl.ds(s * PER_TILE, PER_TILE)])


_sc_agg = pl.kernel(
    _sc_agg_body,
    out_type=jax.ShapeDtypeStruct((NC, ACC_ROWS, HID), jnp.float32),
    mesh=plsc.VectorSubcoreMesh(core_axis_name="c", subcore_axis_name="s",
                                num_cores=NC, num_subcores=NS),
    scratch_types=[
        pltpu.VMEM((HALF_ROWS, IDXW), jnp.int32),
        pltpu.VMEM((HALF_ROWS, IDXW), jnp.int32),
        [pltpu.VMEM((CHUNK, HID), jnp.float32) for _ in range(NBUF)],
        [pltpu.SemaphoreType.DMA for _ in range(NBUF)],
        [pltpu.SemaphoreType.DMA for _ in range(NBUF)],
        pltpu.VMEM_SHARED((ACC_ROWS, HID), jnp.float32),
    ],
)


def _mlp_body(h_ref, p_ref, w1_ref, b1_ref, w2_ref, b2_ref, out_ref):
    a = h_ref[...] + p_ref[0] + p_ref[1]
    t = jnp.dot(a, w1_ref[...], preferred_element_type=jnp.float32) + b1_ref[...]
    t = jnp.maximum(t, 0.0)
    u = jnp.dot(t, w2_ref[...], preferred_element_type=jnp.float32) + b2_ref[...]
    out_ref[...] = jnp.maximum(u, 0.0)


def _mlp_final_body(h_ref, p_ref, w1_ref, b1_ref, w2_ref, b2_ref,
                    wc_ref, bc_ref, out_ref):
    a = h_ref[...] + p_ref[0] + p_ref[1]
    t = jnp.dot(a, w1_ref[...], preferred_element_type=jnp.float32) + b1_ref[...]
    t = jnp.maximum(t, 0.0)
    u = jnp.dot(t, w2_ref[...], preferred_element_type=jnp.float32) + b2_ref[...]
    u = jnp.maximum(u, 0.0)
    out_ref[...] = (jnp.dot(u, wc_ref[...], preferred_element_type=jnp.float32)
                    + bc_ref[...])


def _row_spec(d):
    return pl.BlockSpec((ROWB, d), lambda i: (i, 0))


def _full_spec(shape):
    nd = len(shape)
    return pl.BlockSpec(shape, lambda i: (0,) * nd)


_mlp = pl.pallas_call(
    _mlp_body,
    grid=(N_NODES // ROWB,),
    in_specs=[
        _row_spec(HID),
        pl.BlockSpec((NC, ROWB, HID), lambda i: (0, i, 0)),
        _full_spec((HID, HID)),
        _full_spec((1, HID)),
        _full_spec((HID, HID)),
        _full_spec((1, HID)),
    ],
    out_specs=_row_spec(HID),
    out_shape=jax.ShapeDtypeStruct((N_NODES, HID), jnp.float32),
)

_mlp_final = pl.pallas_call(
    _mlp_final_body,
    grid=(N_NODES // ROWB,),
    in_specs=[
        _row_spec(HID),
        pl.BlockSpec((NC, ROWB, HID), lambda i: (0, i, 0)),
        _full_spec((HID, HID)),
        _full_spec((1, HID)),
        _full_spec((HID, HID)),
        _full_spec((1, HID)),
        _full_spec((HID, OUT_CH)),
        _full_spec((1, OUT_CH)),
    ],
    out_specs=_row_spec(OUT_CH),
    out_shape=jax.ShapeDtypeStruct((N_NODES, OUT_CH), jnp.float32),
)


def kernel(x, edge_index, W1_0, b1_0, W2_0, b2_0, W1_1, b1_1, W2_1, b2_1,
           W1_2, b1_2, W2_2, b2_2, Wc, bc):
    ei = edge_index.astype(jnp.int32)
    pad = E_PAD - N_EDGES
    pad_src = jnp.arange(pad, dtype=jnp.int32) % N_NODES
    pad_dst = DUMMY_ROW + jnp.arange(pad, dtype=jnp.int32) % (ACC_ROWS - N_NODES)
    src_p = jnp.concatenate([ei[0], pad_src]).reshape(NW, IDX_ROWS, IDXW)
    dst_p = jnp.concatenate([ei[1], pad_dst]).reshape(NW, IDX_ROWS, IDXW)
    src_p, dst_p = lax.optimization_barrier((src_p, dst_p))

    params = [(W1_0, b1_0, W2_0, b2_0), (W1_1, b1_1, W2_1, b2_1),
              (W1_2, b1_2, W2_2, b2_2)]
    h = x
    for i, (W1, b1, W2, b2) in enumerate(params):
        part = _sc_agg(h, src_p, dst_p)
        if i < 2:
            h = _mlp(h, part, W1, b1.reshape(1, HID), W2, b2.reshape(1, HID))
        else:
            out = _mlp_final(h, part, W1, b1.reshape(1, HID), W2,
                             b2.reshape(1, HID), Wc, bc.reshape(1, OUT_CH))
    return out


# speedup vs baseline: 12.3591x; 1.0013x over previous
import jax
import jax.numpy as jnp
from jax import lax
from jax.experimental import pallas as pl
from jax.experimental.pallas import tpu as pltpu
from jax.experimental.pallas import tpu_sc as plsc

N_NODES = 10000
N_EDGES = 320000
HID = 128
OUT_CH = 64

NC, NS = 2, 16
NW = NC * NS
CHUNK = 64
IDXW = 128
IDX_ROWS = 80
HALF_ROWS = IDX_ROWS // 2
HALF_CHUNKS = HALF_ROWS * 2
E_PAD = NW * IDX_ROWS * IDXW
NBUF = 4
ACC_ROWS = 10112
DUMMY_ROW = N_NODES
PER_TILE = ACC_ROWS // NS

ROWB = 2000


def _sc_agg_body(h_hbm, isrc_hbm, idst_hbm, part_hbm,
                 isrc_v, idst_v, rows, sem_g, sem_s, acc_sh):
    c = lax.axis_index("c")
    s = lax.axis_index("s")
    wid = c * NS + s

    def _sidx(ref, k, b):
        return ref.at[2 * k + (b // 2), pl.ds(64 * (b % 2), 64)]

    pltpu.sync_copy(isrc_hbm.at[wid, pl.ds(0, HALF_ROWS)], isrc_v)
    pltpu.sync_copy(idst_hbm.at[wid, pl.ds(0, HALF_ROWS)], idst_v)
    for b in range(NBUF - 1):
        pltpu.async_copy(h_hbm.at[_sidx(isrc_v, 0, b)], rows[b], sem_g[b])

    zbuf = rows[NBUF - 1]

    def _zrow(i, carry):
        for k in range(HID // 16):
            zbuf[i, pl.ds(k * 16, 16)] = jnp.zeros((16,), jnp.float32)
        return carry
    lax.fori_loop(0, CHUNK, _zrow, 0)
    base = s * PER_TILE
    nz = PER_TILE // CHUNK
    rem = PER_TILE % CHUNK
    for r in range(nz):
        pltpu.async_copy(
            zbuf, acc_sh.at[pl.ds(base + r * CHUNK, CHUNK)], sem_s[0])
    if rem:
        pltpu.async_copy(
            zbuf.at[pl.ds(0, rem)],
            acc_sh.at[pl.ds(base + nz * CHUNK, rem)], sem_s[0])
    for r in range(nz):
        pltpu.make_async_copy(
            zbuf, acc_sh.at[pl.ds(base + r * CHUNK, CHUNK)], sem_s[0]).wait()
    if rem:
        pltpu.make_async_copy(
            zbuf.at[pl.ds(0, rem)],
            acc_sh.at[pl.ds(base + nz * CHUNK, rem)], sem_s[0]).wait()
    pltpu.async_copy(
        h_hbm.at[_sidx(isrc_v, 0, NBUF - 1)], rows[NBUF - 1], sem_g[NBUF - 1])
    plsc.subcore_barrier()

    for half in range(2):
        if half:
            pltpu.sync_copy(
                isrc_hbm.at[wid, pl.ds(HALF_ROWS, HALF_ROWS)], isrc_v)
            pltpu.sync_copy(
                idst_hbm.at[wid, pl.ds(HALF_ROWS, HALF_ROWS)], idst_v)
            for b in range(NBUF):
                pltpu.async_copy(
                    h_hbm.at[_sidx(isrc_v, 0, b)], rows[b], sem_g[b])

        def _round(k, carry):
            for b in range(NBUF):
                pltpu.make_async_copy(
                    h_hbm.at[_sidx(isrc_v, k, b)], rows[b], sem_g[b]).wait()
                pltpu.async_copy(
                    rows[b], acc_sh.at[_sidx(idst_v, k, b)], sem_s[b],
                    add=True)
            for b in range(NBUF):
                @pl.when(NBUF * k + b + NBUF < HALF_CHUNKS)
                def _():
                    pltpu.make_async_copy(
                        rows[b], acc_sh.at[_sidx(idst_v, k, b)],
                        sem_s[b]).wait()
                    pltpu.async_copy(
                        h_hbm.at[_sidx(isrc_v, k + 1, b)], rows[b], sem_g[b])
            return carry

        lax.fori_loop(0, HALF_CHUNKS // NBUF, _round, 0)
        last = HALF_CHUNKS // NBUF - 1
        for b in range(NBUF):
            pltpu.make_async_copy(
                rows[b], acc_sh.at[_sidx(idst_v, last, b)], sem_s[b]).wait()
    plsc.subcore_barrier()

    pltpu.sync_copy(acc_sh.at[pl.ds(s * PER_TILE, PER_TILE)],
                    part_hbm.at[c, pl.ds(s * PER_TILE, PER_TILE)])


_sc_agg = pl.kernel(
    _sc_agg_body,
    out_type=jax.ShapeDtypeStruct((NC, ACC_ROWS, HID), jnp.float32),
    mesh=plsc.VectorSubcoreMesh(core_axis_name="c", subcore_axis_name="s",
                                num_cores=NC, num_subcores=NS),
    scratch_types=[
        pltpu.VMEM((HALF_ROWS, IDXW), jnp.int32),
        pltpu.VMEM((HALF_ROWS, IDXW), jnp.int32),
        [pltpu.VMEM((CHUNK, HID), jnp.float32) for _ in range(NBUF)],
        [pltpu.SemaphoreType.DMA for _ in range(NBUF)],
        [pltpu.SemaphoreType.DMA for _ in range(NBUF)],
        pltpu.VMEM_SHARED((ACC_ROWS, HID), jnp.float32),
    ],
)


def _mlp_body(h_ref, p_ref, w1_ref, b1_ref, w2_ref, b2_ref, out_ref):
    a = h_ref[...] + p_ref[0] + p_ref[1]
    t = jnp.dot(a, w1_ref[...], preferred_element_type=jnp.float32) + b1_ref[...]
    t = jnp.maximum(t, 0.0)
    u = jnp.dot(t, w2_ref[...], preferred_element_type=jnp.float32) + b2_ref[...]
    out_ref[...] = jnp.maximum(u, 0.0)


def _mlp_final_body(h_ref, p_ref, w1_ref, b1_ref, w2_ref, b2_ref,
                    wc_ref, bc_ref, out_ref):
    a = h_ref[...] + p_ref[0] + p_ref[1]
    t = jnp.dot(a, w1_ref[...], preferred_element_type=jnp.float32) + b1_ref[...]
    t = jnp.maximum(t, 0.0)
    u = jnp.dot(t, w2_ref[...], preferred_element_type=jnp.float32) + b2_ref[...]
    u = jnp.maximum(u, 0.0)
    out_ref[...] = (jnp.dot(u, wc_ref[...], preferred_element_type=jnp.float32)
                    + bc_ref[...])


def _row_spec(d):
    return pl.BlockSpec((ROWB, d), lambda i: (i, 0))


def _full_spec(shape):
    nd = len(shape)
    return pl.BlockSpec(shape, lambda i: (0,) * nd)


_mlp = pl.pallas_call(
    _mlp_body,
    grid=(N_NODES // ROWB,),
    in_specs=[
        _row_spec(HID),
        pl.BlockSpec((NC, ROWB, HID), lambda i: (0, i, 0)),
        _full_spec((HID, HID)),
        _full_spec((1, HID)),
        _full_spec((HID, HID)),
        _full_spec((1, HID)),
    ],
    out_specs=_row_spec(HID),
    out_shape=jax.ShapeDtypeStruct((N_NODES, HID), jnp.float32),
)

_mlp_final = pl.pallas_call(
    _mlp_final_body,
    grid=(N_NODES // ROWB,),
    in_specs=[
        _row_spec(HID),
        pl.BlockSpec((NC, ROWB, HID), lambda i: (0, i, 0)),
        _full_spec((HID, HID)),
        _full_spec((1, HID)),
        _full_spec((HID, HID)),
        _full_spec((1, HID)),
        _full_spec((HID, OUT_CH)),
        _full_spec((1, OUT_CH)),
    ],
    out_specs=_row_spec(OUT_CH),
    out_shape=jax.ShapeDtypeStruct((N_NODES, OUT_CH), jnp.float32),
)


def kernel(x, edge_index, W1_0, b1_0, W2_0, b2_0, W1_1, b1_1, W2_1, b2_1,
           W1_2, b1_2, W2_2, b2_2, Wc, bc):
    ei = edge_index.astype(jnp.int32)
    pad = E_PAD - N_EDGES
    pad_src = jnp.arange(pad, dtype=jnp.int32) & 8191
    pad_dst = DUMMY_ROW + (jnp.arange(pad, dtype=jnp.int32) & 63)
    src_p = jnp.concatenate([ei[0], pad_src]).reshape(NW, IDX_ROWS, IDXW)
    dst_p = jnp.concatenate([ei[1], pad_dst]).reshape(NW, IDX_ROWS, IDXW)
    src_p, dst_p = lax.optimization_barrier((src_p, dst_p))

    params = [(W1_0, b1_0, W2_0, b2_0), (W1_1, b1_1, W2_1, b2_1),
              (W1_2, b1_2, W2_2, b2_2)]
    h = x
    for i, (W1, b1, W2, b2) in enumerate(params):
        part = _sc_agg(h, src_p, dst_p)
        if i < 2:
            h = _mlp(h, part, W1, b1.reshape(1, HID), W2, b2.reshape(1, HID))
        else:
            out = _mlp_final(h, part, W1, b1.reshape(1, HID), W2,
                             b2.reshape(1, HID), Wc, bc.reshape(1, OUT_CH))
    return out


# speedup vs baseline: 12.6017x; 1.0196x over previous
import jax
import jax.numpy as jnp
from jax import lax
from jax.experimental import pallas as pl
from jax.experimental.pallas import tpu as pltpu
from jax.experimental.pallas import tpu_sc as plsc

N_NODES = 10000
N_EDGES = 320000
HID = 128
OUT_CH = 64

NC, NS = 2, 16
NW = NC * NS
CHUNK = 32
IDXW = 128
CPR = IDXW // CHUNK
IDX_ROWS = 80
HALF_ROWS = IDX_ROWS // 2
HALF_CHUNKS = HALF_ROWS * CPR
E_PAD = NW * IDX_ROWS * IDXW
NBUF = 8
ACC_ROWS = 10112
DUMMY_ROW = N_NODES
PER_TILE = ACC_ROWS // NS

ROWB = 2000


def _sc_agg_body(h_hbm, isrc_hbm, idst_hbm, part_hbm,
                 isrc_v, idst_v, rows, sem_g, sem_s, acc_sh):
    c = lax.axis_index("c")
    s = lax.axis_index("s")
    wid = c * NS + s

    def _sidx(ref, k, b):
        return ref.at[(NBUF // CPR) * k + (b // CPR),
                      pl.ds(CHUNK * (b % CPR), CHUNK)]

    pltpu.sync_copy(isrc_hbm.at[wid, pl.ds(0, HALF_ROWS)], isrc_v)
    pltpu.sync_copy(idst_hbm.at[wid, pl.ds(0, HALF_ROWS)], idst_v)
    for b in range(NBUF - 1):
        pltpu.async_copy(h_hbm.at[_sidx(isrc_v, 0, b)], rows[b], sem_g[b])

    zbuf = rows[NBUF - 1]

    def _zrow(i, carry):
        for k in range(HID // 16):
            zbuf[i, pl.ds(k * 16, 16)] = jnp.zeros((16,), jnp.float32)
        return carry
    lax.fori_loop(0, CHUNK, _zrow, 0)
    base = s * PER_TILE
    nz = PER_TILE // CHUNK
    rem = PER_TILE % CHUNK
    for r in range(nz):
        pltpu.async_copy(
            zbuf, acc_sh.at[pl.ds(base + r * CHUNK, CHUNK)], sem_s[0])
    if rem:
        pltpu.async_copy(
            zbuf.at[pl.ds(0, rem)],
            acc_sh.at[pl.ds(base + nz * CHUNK, rem)], sem_s[0])
    for r in range(nz):
        pltpu.make_async_copy(
            zbuf, acc_sh.at[pl.ds(base + r * CHUNK, CHUNK)], sem_s[0]).wait()
    if rem:
        pltpu.make_async_copy(
            zbuf.at[pl.ds(0, rem)],
            acc_sh.at[pl.ds(base + nz * CHUNK, rem)], sem_s[0]).wait()
    pltpu.async_copy(
        h_hbm.at[_sidx(isrc_v, 0, NBUF - 1)], rows[NBUF - 1], sem_g[NBUF - 1])
    plsc.subcore_barrier()

    for half in range(2):
        if half:
            pltpu.sync_copy(
                isrc_hbm.at[wid, pl.ds(HALF_ROWS, HALF_ROWS)], isrc_v)
            pltpu.sync_copy(
                idst_hbm.at[wid, pl.ds(HALF_ROWS, HALF_ROWS)], idst_v)
            for b in range(NBUF):
                pltpu.async_copy(
                    h_hbm.at[_sidx(isrc_v, 0, b)], rows[b], sem_g[b])

        def _round(k, carry):
            for b in range(NBUF):
                pltpu.make_async_copy(
                    h_hbm.at[_sidx(isrc_v, k, b)], rows[b], sem_g[b]).wait()
                pltpu.async_copy(
                    rows[b], acc_sh.at[_sidx(idst_v, k, b)], sem_s[b],
                    add=True)
            for b in range(NBUF):
                @pl.when(NBUF * k + b + NBUF < HALF_CHUNKS)
                def _():
                    pltpu.make_async_copy(
                        rows[b], acc_sh.at[_sidx(idst_v, k, b)],
                        sem_s[b]).wait()
                    pltpu.async_copy(
                        h_hbm.at[_sidx(isrc_v, k + 1, b)], rows[b], sem_g[b])
            return carry

        lax.fori_loop(0, HALF_CHUNKS // NBUF, _round, 0)
        last = HALF_CHUNKS // NBUF - 1
        for b in range(NBUF):
            pltpu.make_async_copy(
                rows[b], acc_sh.at[_sidx(idst_v, last, b)], sem_s[b]).wait()
    plsc.subcore_barrier()

    pltpu.sync_copy(acc_sh.at[pl.ds(s * PER_TILE, PER_TILE)],
                    part_hbm.at[c, pl.ds(s * PER_TILE, PER_TILE)])


_sc_agg = pl.kernel(
    _sc_agg_body,
    out_type=jax.ShapeDtypeStruct((NC, ACC_ROWS, HID), jnp.float32),
    mesh=plsc.VectorSubcoreMesh(core_axis_name="c", subcore_axis_name="s",
                                num_cores=NC, num_subcores=NS),
    scratch_types=[
        pltpu.VMEM((HALF_ROWS, IDXW), jnp.int32),
        pltpu.VMEM((HALF_ROWS, IDXW), jnp.int32),
        [pltpu.VMEM((CHUNK, HID), jnp.float32) for _ in range(NBUF)],
        [pltpu.SemaphoreType.DMA for _ in range(NBUF)],
        [pltpu.SemaphoreType.DMA for _ in range(NBUF)],
        pltpu.VMEM_SHARED((ACC_ROWS, HID), jnp.float32),
    ],
)


def _mlp_body(h_ref, p_ref, w1_ref, b1_ref, w2_ref, b2_ref, out_ref):
    a = h_ref[...] + p_ref[0] + p_ref[1]
    t = jnp.dot(a, w1_ref[...], preferred_element_type=jnp.float32) + b1_ref[...]
    t = jnp.maximum(t, 0.0)
    u = jnp.dot(t, w2_ref[...], preferred_element_type=jnp.float32) + b2_ref[...]
    out_ref[...] = jnp.maximum(u, 0.0)


def _mlp_final_body(h_ref, p_ref, w1_ref, b1_ref, w2_ref, b2_ref,
                    wc_ref, bc_ref, out_ref):
    a = h_ref[...] + p_ref[0] + p_ref[1]
    t = jnp.dot(a, w1_ref[...], preferred_element_type=jnp.float32) + b1_ref[...]
    t = jnp.maximum(t, 0.0)
    u = jnp.dot(t, w2_ref[...], preferred_element_type=jnp.float32) + b2_ref[...]
    u = jnp.maximum(u, 0.0)
    out_ref[...] = (jnp.dot(u, wc_ref[...], preferred_element_type=jnp.float32)
                    + bc_ref[...])


def _row_spec(d):
    return pl.BlockSpec((ROWB, d), lambda i: (i, 0))


def _full_spec(shape):
    nd = len(shape)
    return pl.BlockSpec(shape, lambda i: (0,) * nd)


_mlp = pl.pallas_call(
    _mlp_body,
    grid=(N_NODES // ROWB,),
    in_specs=[
        _row_spec(HID),
        pl.BlockSpec((NC, ROWB, HID), lambda i: (0, i, 0)),
        _full_spec((HID, HID)),
        _full_spec((1, HID)),
        _full_spec((HID, HID)),
        _full_spec((1, HID)),
    ],
    out_specs=_row_spec(HID),
    out_shape=jax.ShapeDtypeStruct((N_NODES, HID), jnp.float32),
)

_mlp_final = pl.pallas_call(
    _mlp_final_body,
    grid=(N_NODES // ROWB,),
    in_specs=[
        _row_spec(HID),
        pl.BlockSpec((NC, ROWB, HID), lambda i: (0, i, 0)),
        _full_spec((HID, HID)),
        _full_spec((1, HID)),
        _full_spec((HID, HID)),
        _full_spec((1, HID)),
        _full_spec((HID, OUT_CH)),
        _full_spec((1, OUT_CH)),
    ],
    out_specs=_row_spec(OUT_CH),
    out_shape=jax.ShapeDtypeStruct((N_NODES, OUT_CH), jnp.float32),
)


def kernel(x, edge_index, W1_0, b1_0, W2_0, b2_0, W1_1, b1_1, W2_1, b2_1,
           W1_2, b1_2, W2_2, b2_2, Wc, bc):
    ei = edge_index.astype(jnp.int32)
    pad = E_PAD - N_EDGES
    pad_src = jnp.arange(pad, dtype=jnp.int32) & 8191
    pad_dst = DUMMY_ROW + (jnp.arange(pad, dtype=jnp.int32) & 63)
    src_p = jnp.concatenate([ei[0], pad_src]).reshape(NW, IDX_ROWS, IDXW)
    dst_p = jnp.concatenate([ei[1], pad_dst]).reshape(NW, IDX_ROWS, IDXW)
    src_p, dst_p = lax.optimization_barrier((src_p, dst_p))

    params = [(W1_0, b1_0, W2_0, b2_0), (W1_1, b1_1, W2_1, b2_1),
              (W1_2, b1_2, W2_2, b2_2)]
    h = x
    for i, (W1, b1, W2, b2) in enumerate(params):
        part = _sc_agg(h, src_p, dst_p)
        if i < 2:
            h = _mlp(h, part, W1, b1.reshape(1, HID), W2, b2.reshape(1, HID))
        else:
            out = _mlp_final(h, part, W1, b1.reshape(1, HID), W2,
                             b2.reshape(1, HID), Wc, bc.reshape(1, OUT_CH))
    return out
